# baseline (speedup 1.0000x reference)
import functools

import jax
import jax.numpy as jnp
from jax import lax
from jax.experimental import pallas as pl
from jax.experimental.pallas import tpu as pltpu
from jax.experimental.pallas import tpu_sc as plsc

_H, _B, _N, _D = 1, 16, 576, 256
_K = 8192
_BN = _B * _N
_TILE_M = 256
_M_TILES = _BN // _TILE_M

_NC, _NS = 2, 16
_NW = _NC * _NS
_BPW = _BN // _NW
_CHUNK = 96
_NCHUNK = _BPW // _CHUNK


def _dist_body(x_ref, e_ref, x2_ref, y2_ref, dist_ref, ind_ref):
    xt = x_ref[...]
    e = e_ref[...]
    x2 = x2_ref[...]
    y2 = y2_ref[...]
    xy = lax.dot_general(
        xt, e, (((1,), (1,)), ((), ())),
        preferred_element_type=jnp.float32,
        precision=lax.Precision.DEFAULT) * -2.0
    sq = (x2[:, None] + y2[None, :]) + xy
    dist = -jnp.sqrt(jnp.maximum(sq, 0.0))
    dist_ref[...] = dist
    vals = dist
    idxs = lax.broadcasted_iota(jnp.int32, (_TILE_M, _K), 1)
    w = _K
    while w > 128:
        h = w // 2
        a, b = vals[:, :h], vals[:, h:w]
        keep = a >= b
        vals = jnp.where(keep, a, b)
        idxs = jnp.where(keep, idxs[:, :h], idxs[:, h:w])
        w = h
    mx = jnp.max(vals, axis=1, keepdims=True)
    ind_ref[...] = jnp.min(jnp.where(vals == mx, idxs, _K), axis=1)


def _distances(x_flat, table, x2, y2):
    return pl.pallas_call(
        _dist_body,
        grid=(_M_TILES,),
        in_specs=[
            pl.BlockSpec((_TILE_M, _D), lambda i: (i, 0)),
            pl.BlockSpec((_K, _D), lambda i: (0, 0)),
            pl.BlockSpec((_TILE_M,), lambda i: (i,)),
            pl.BlockSpec((_K,), lambda i: (0,)),
        ],
        out_specs=[
            pl.BlockSpec((_TILE_M, _K), lambda i: (i, 0)),
            pl.BlockSpec((_TILE_M,), lambda i: (i,)),
        ],
        out_shape=[
            jax.ShapeDtypeStruct((_BN, _K), jnp.float32),
            jax.ShapeDtypeStruct((_BN,), jnp.int32),
        ],
    )(x_flat, table, x2, y2)


@functools.cache
def _make_sc_gather():
    @functools.partial(
        pl.kernel,
        mesh=plsc.VectorSubcoreMesh(core_axis_name="c", subcore_axis_name="s"),
        out_type=jax.ShapeDtypeStruct((_BN, _D), jnp.float32),
        scratch_types=[
            pltpu.VMEM((_BPW,), jnp.int32),
            pltpu.VMEM((_BPW, _D), jnp.float32),
            pltpu.SemaphoreType.DMA,
        ],
    )
    def _sc_gather(table_hbm, idx_hbm, out_hbm, idx_v, rows_v, sem):
        wid = lax.axis_index("s") * _NC + lax.axis_index("c")
        base = wid * _BPW
        pltpu.sync_copy(idx_hbm.at[pl.ds(base, _BPW)], idx_v)
        copies = [
            pltpu.async_copy(
                table_hbm.at[idx_v.at[pl.ds(c * _CHUNK, _CHUNK)]],
                rows_v.at[pl.ds(c * _CHUNK, _CHUNK)], sem)
            for c in range(_NCHUNK)
        ]
        for cp in copies:
            cp.wait()
        pltpu.sync_copy(rows_v, out_hbm.at[pl.ds(base, _BPW)])

    return _sc_gather


def kernel(x, embed):
    h, b, n, d = x.shape
    k = embed.shape[1]
    x_flat = x.astype(jnp.float32).reshape(h * b * n, d)
    table = embed.reshape(k, d)
    x2 = jnp.sum(x_flat ** 2, axis=-1)
    y2 = jnp.sum(table ** 2, axis=-1)
    dist_flat, ind_flat = _distances(
        x_flat.astype(jnp.bfloat16), table.astype(jnp.bfloat16), x2, y2)
    quantize = _make_sc_gather()(table, ind_flat)
    return (quantize.reshape(h, b, n, d),
            ind_flat.reshape(h, b, n),
            dist_flat.reshape(h, b, n, k))


# speedup vs baseline: 1.0667x; 1.0667x over previous
import functools

import jax
import jax.numpy as jnp
from jax import lax
from jax.experimental import pallas as pl
from jax.experimental.pallas import tpu as pltpu
from jax.experimental.pallas import tpu_sc as plsc

_H, _B, _N, _D = 1, 16, 576, 256
_K = 8192
_BN = _B * _N
_TILE_M = 512
_M_TILES = _BN // _TILE_M

_NC, _NS = 2, 16
_NW = _NC * _NS
_BPW = _BN // _NW
_CHUNK = 96
_NCHUNK = _BPW // _CHUNK


def _dist_body(x_ref, e_ref, x2_ref, y2_ref, dist_ref, ind_ref):
    xt = x_ref[...]
    e = e_ref[...]
    x2 = x2_ref[...]
    y2 = y2_ref[...]
    xy = lax.dot_general(
        xt, e, (((1,), (1,)), ((), ())),
        preferred_element_type=jnp.float32,
        precision=lax.Precision.DEFAULT) * -2.0
    sq = (x2[:, None] + y2[None, :]) + xy
    dist = -jnp.sqrt(jnp.maximum(sq, 0.0))
    dist_ref[...] = dist
    mx = jnp.max(dist, axis=1, keepdims=True)
    iota = lax.broadcasted_iota(jnp.int32, (_TILE_M, _K), 1)
    ind_ref[...] = jnp.min(jnp.where(dist == mx, iota, _K), axis=1)


def _distances(x_flat, table, x2, y2):
    return pl.pallas_call(
        _dist_body,
        grid=(_M_TILES,),
        in_specs=[
            pl.BlockSpec((_TILE_M, _D), lambda i: (i, 0)),
            pl.BlockSpec((_K, _D), lambda i: (0, 0)),
            pl.BlockSpec((_TILE_M,), lambda i: (i,)),
            pl.BlockSpec((_K,), lambda i: (0,)),
        ],
        out_specs=[
            pl.BlockSpec((_TILE_M, _K), lambda i: (i, 0)),
            pl.BlockSpec((_TILE_M,), lambda i: (i,)),
        ],
        out_shape=[
            jax.ShapeDtypeStruct((_BN, _K), jnp.float32),
            jax.ShapeDtypeStruct((_BN,), jnp.int32),
        ],
    )(x_flat, table, x2, y2)


@functools.cache
def _make_sc_gather():
    @functools.partial(
        pl.kernel,
        mesh=plsc.VectorSubcoreMesh(core_axis_name="c", subcore_axis_name="s"),
        out_type=jax.ShapeDtypeStruct((_BN, _D), jnp.float32),
        scratch_types=[
            pltpu.VMEM((_BPW,), jnp.int32),
            pltpu.VMEM((_BPW, _D), jnp.float32),
            pltpu.SemaphoreType.DMA,
        ],
    )
    def _sc_gather(table_hbm, idx_hbm, out_hbm, idx_v, rows_v, sem):
        wid = lax.axis_index("s") * _NC + lax.axis_index("c")
        base = wid * _BPW
        pltpu.sync_copy(idx_hbm.at[pl.ds(base, _BPW)], idx_v)
        copies = [
            pltpu.async_copy(
                table_hbm.at[idx_v.at[pl.ds(c * _CHUNK, _CHUNK)]],
                rows_v.at[pl.ds(c * _CHUNK, _CHUNK)], sem)
            for c in range(_NCHUNK)
        ]
        for cp in copies:
            cp.wait()
        pltpu.sync_copy(rows_v, out_hbm.at[pl.ds(base, _BPW)])

    return _sc_gather


def kernel(x, embed):
    h, b, n, d = x.shape
    k = embed.shape[1]
    x_flat = x.astype(jnp.float32).reshape(h * b * n, d)
    table = embed.reshape(k, d)
    x2 = jnp.sum(x_flat ** 2, axis=-1)
    y2 = jnp.sum(table ** 2, axis=-1)
    dist_flat, ind_flat = _distances(
        x_flat.astype(jnp.bfloat16), table.astype(jnp.bfloat16), x2, y2)
    quantize = _make_sc_gather()(table, ind_flat)
    return (quantize.reshape(h, b, n, d),
            ind_flat.reshape(h, b, n),
            dist_flat.reshape(h, b, n, k))


# speedup vs baseline: 1.1569x; 1.0845x over previous
import functools

import jax
import jax.numpy as jnp
from jax import lax
from jax.experimental import pallas as pl
from jax.experimental.pallas import tpu as pltpu
from jax.experimental.pallas import tpu_sc as plsc

_H, _B, _N, _D = 1, 16, 576, 256
_K = 8192
_BN = _B * _N
_TILE_M = 512
_M_TILES = _BN // _TILE_M

_NC, _NS = 2, 16
_NW = _NC * _NS
_BPW = _BN // _NW
_CHUNK = 96
_NCHUNK = _BPW // _CHUNK


def _dist_body(x_ref, e_ref, x2_ref, y2_ref, dist_ref, ind_ref):
    xt = x_ref[...]
    e = e_ref[...]
    x2 = x2_ref[...]
    y2 = y2_ref[...]
    xy = lax.dot_general(
        xt, e, (((1,), (1,)), ((), ())),
        preferred_element_type=jnp.float32,
        precision=lax.Precision.DEFAULT) * -2.0
    sq = (x2[:, None] + y2[None, :]) + xy
    dist = -jnp.sqrt(jnp.maximum(sq, 0.0))
    dist_ref[...] = dist
    ind_ref[...] = jnp.argmax(dist, axis=1).astype(jnp.int32)


def _distances(x_flat, table, x2, y2):
    return pl.pallas_call(
        _dist_body,
        grid=(_M_TILES,),
        in_specs=[
            pl.BlockSpec((_TILE_M, _D), lambda i: (i, 0)),
            pl.BlockSpec((_K, _D), lambda i: (0, 0)),
            pl.BlockSpec((_TILE_M,), lambda i: (i,)),
            pl.BlockSpec((_K,), lambda i: (0,)),
        ],
        out_specs=[
            pl.BlockSpec((_TILE_M, _K), lambda i: (i, 0)),
            pl.BlockSpec((_TILE_M,), lambda i: (i,)),
        ],
        out_shape=[
            jax.ShapeDtypeStruct((_BN, _K), jnp.float32),
            jax.ShapeDtypeStruct((_BN,), jnp.int32),
        ],
    )(x_flat, table, x2, y2)


@functools.cache
def _make_sc_gather():
    @functools.partial(
        pl.kernel,
        mesh=plsc.VectorSubcoreMesh(core_axis_name="c", subcore_axis_name="s"),
        out_type=jax.ShapeDtypeStruct((_BN, _D), jnp.float32),
        scratch_types=[
            pltpu.VMEM((_BPW,), jnp.int32),
            pltpu.VMEM((_BPW, _D), jnp.float32),
            pltpu.SemaphoreType.DMA,
        ],
    )
    def _sc_gather(table_hbm, idx_hbm, out_hbm, idx_v, rows_v, sem):
        wid = lax.axis_index("s") * _NC + lax.axis_index("c")
        base = wid * _BPW
        pltpu.sync_copy(idx_hbm.at[pl.ds(base, _BPW)], idx_v)
        copies = [
            pltpu.async_copy(
                table_hbm.at[idx_v.at[pl.ds(c * _CHUNK, _CHUNK)]],
                rows_v.at[pl.ds(c * _CHUNK, _CHUNK)], sem)
            for c in range(_NCHUNK)
        ]
        for cp in copies:
            cp.wait()
        pltpu.sync_copy(rows_v, out_hbm.at[pl.ds(base, _BPW)])

    return _sc_gather


def kernel(x, embed):
    h, b, n, d = x.shape
    k = embed.shape[1]
    x_flat = x.astype(jnp.float32).reshape(h * b * n, d)
    table = embed.reshape(k, d)
    x2 = jnp.sum(x_flat ** 2, axis=-1)
    y2 = jnp.sum(table ** 2, axis=-1)
    dist_flat, ind_flat = _distances(
        x_flat.astype(jnp.bfloat16), table.astype(jnp.bfloat16), x2, y2)
    quantize = _make_sc_gather()(table, ind_flat)
    return (quantize.reshape(h, b, n, d),
            ind_flat.reshape(h, b, n),
            dist_flat.reshape(h, b, n, k))


# speedup vs baseline: 1.1897x; 1.0284x over previous
import functools

import jax
import jax.numpy as jnp
from jax import lax
from jax.experimental import pallas as pl
from jax.experimental.pallas import tpu as pltpu
from jax.experimental.pallas import tpu_sc as plsc

_H, _B, _N, _D = 1, 16, 576, 256
_K = 8192
_BN = _B * _N
_TILE_M = 512
_M_TILES = _BN // _TILE_M

_NC, _NS = 2, 16
_NW = _NC * _NS
_BPW = _BN // _NW
_CHUNK = 96
_NCHUNK = _BPW // _CHUNK


def _dist_body(x_ref, e_ref, x2_ref, y2_ref, dist_ref, ind_ref):
    xt = x_ref[...]
    e = e_ref[...]
    x2 = x2_ref[...]
    y2 = y2_ref[...]
    xy = lax.dot_general(
        xt, e, (((1,), (1,)), ((), ())),
        preferred_element_type=jnp.float32,
        precision=lax.Precision.DEFAULT)
    sq = (x2[:, None] + y2[None, :]) + xy
    dist = -jnp.sqrt(jnp.maximum(sq, 0.0))
    dist_ref[...] = dist
    ind_ref[...] = jnp.argmax(dist, axis=1).astype(jnp.int32)


def _distances(x_flat, table, x2, y2):
    return pl.pallas_call(
        _dist_body,
        grid=(_M_TILES,),
        in_specs=[
            pl.BlockSpec((_TILE_M, _D), lambda i: (i, 0)),
            pl.BlockSpec((_K, _D), lambda i: (0, 0)),
            pl.BlockSpec((_TILE_M,), lambda i: (i,)),
            pl.BlockSpec((_K,), lambda i: (0,)),
        ],
        out_specs=[
            pl.BlockSpec((_TILE_M, _K), lambda i: (i, 0)),
            pl.BlockSpec((_TILE_M,), lambda i: (i,)),
        ],
        out_shape=[
            jax.ShapeDtypeStruct((_BN, _K), jnp.float32),
            jax.ShapeDtypeStruct((_BN,), jnp.int32),
        ],
    )(x_flat, table, x2, y2)


@functools.cache
def _make_sc_gather():
    @functools.partial(
        pl.kernel,
        mesh=plsc.VectorSubcoreMesh(core_axis_name="c", subcore_axis_name="s"),
        out_type=jax.ShapeDtypeStruct((_BN, _D), jnp.float32),
        scratch_types=[
            pltpu.VMEM((_BPW,), jnp.int32),
            pltpu.VMEM((_BPW, _D), jnp.float32),
            pltpu.SemaphoreType.DMA,
        ],
    )
    def _sc_gather(table_hbm, idx_hbm, out_hbm, idx_v, rows_v, sem):
        wid = lax.axis_index("s") * _NC + lax.axis_index("c")
        base = wid * _BPW
        pltpu.sync_copy(idx_hbm.at[pl.ds(base, _BPW)], idx_v)
        copies = [
            pltpu.async_copy(
                table_hbm.at[idx_v.at[pl.ds(c * _CHUNK, _CHUNK)]],
                rows_v.at[pl.ds(c * _CHUNK, _CHUNK)], sem)
            for c in range(_NCHUNK)
        ]
        for cp in copies:
            cp.wait()
        pltpu.sync_copy(rows_v, out_hbm.at[pl.ds(base, _BPW)])

    return _sc_gather


def kernel(x, embed):
    h, b, n, d = x.shape
    k = embed.shape[1]
    x_flat = x.astype(jnp.float32).reshape(h * b * n, d)
    table = embed.reshape(k, d)
    x2 = jnp.sum(x_flat ** 2, axis=-1)
    y2 = jnp.sum(table ** 2, axis=-1)
    dist_flat, ind_flat = _distances(
        (x_flat * -2.0).astype(jnp.bfloat16), table.astype(jnp.bfloat16),
        x2, y2)
    quantize = _make_sc_gather()(table, ind_flat)
    return (quantize.reshape(h, b, n, d),
            ind_flat.reshape(h, b, n),
            dist_flat.reshape(h, b, n, k))
